# baseline (speedup 1.0000x reference)
import functools

import jax
import jax.numpy as jnp
from jax import lax
from jax.experimental import pallas as pl
from jax.experimental.pallas import tpu as pltpu
from jax.experimental.pallas import tpu_sc as plsc

N_DENSE = 13
N_SPARSE = 26
VOCAB = 100000
EMBED = 16
BATCH = 4096
H1 = 256
H2 = 256
BN_EPS = 1e-3

NC = 2
NS = 16
NW = NC * NS
BPW = BATCH // NW
D_EMB = N_SPARSE * EMBED
D_PAD = 512
PACK = 128 // EMBED
RPF = VOCAB // PACK
NBLK = D_PAD // 128


def _sc_gather(tab128, row3, sub3):
    mesh = plsc.VectorSubcoreMesh(core_axis_name="c", subcore_axis_name="s")

    @functools.partial(
        pl.kernel,
        out_type=jax.ShapeDtypeStruct((BATCH, D_PAD), jnp.float32),
        mesh=mesh,
        scratch_types=[
            pltpu.VMEM((N_SPARSE, BPW), jnp.int32),
            pltpu.VMEM((N_SPARSE, BPW), jnp.int32),
            pltpu.VMEM((BPW, 128), jnp.float32),
            pltpu.VMEM((BPW, 128), jnp.float32),
            pltpu.VMEM((NBLK, BPW, 128), jnp.float32),
            pltpu.SemaphoreType.DMA,
            pltpu.SemaphoreType.DMA,
        ],
        compiler_params=pltpu.CompilerParams(needs_layout_passes=False),
    )
    def gather_kernel(tab_hbm, row_hbm, sub_hbm, out_hbm,
                      row_v, sub_v, buf0, buf1, emb_v, sem0, sem1):
        wid = lax.axis_index("s") * NC + lax.axis_index("c")
        pltpu.sync_copy(row_hbm.at[wid], row_v)
        pltpu.sync_copy(sub_hbm.at[wid], sub_v)

        zero16 = jnp.zeros((16,), jnp.float32)

        def zero_tail(u, c):
            for k in range(D_EMB - 3 * 128, 128, 16):
                emb_v[3, u, pl.ds(k, 16)] = zero16
            return c
        lax.fori_loop(0, BPW, zero_tail, 0)

        def fire(f, buf, sem):
            return pltpu.async_copy(tab_hbm.at[row_v.at[f]], buf, sem)

        def wait(f, buf, sem):
            pltpu.make_async_copy(tab_hbm.at[row_v.at[f]], buf, sem).wait()

        def extract(f, buf):
            jj = f // PACK
            cbase = (f % PACK) * EMBED
            for g in range(BPW // 16):
                rows = lax.iota(jnp.int32, 16) + (g * 16)
                sub = sub_v[f, pl.ds(g * 16, 16)]
                col0 = sub * EMBED
                for e in range(EMBED):
                    vals = plsc.load_gather(buf, [rows, col0 + e])
                    dcol = jnp.full((16,), cbase + e, jnp.int32)
                    plsc.store_scatter(emb_v.at[jj], [rows, dcol], vals)

        fire(0, buf0, sem0)
        fire(1, buf1, sem1)

        def body(k, c):
            f0 = 2 * k
            f1 = 2 * k + 1
            wait(f0, buf0, sem0)
            extract(f0, buf0)

            @pl.when(k < N_SPARSE // 2 - 1)
            def _():
                fire(f0 + 2, buf0, sem0)
            wait(f1, buf1, sem1)
            extract(f1, buf1)

            @pl.when(k < N_SPARSE // 2 - 1)
            def _():
                fire(f1 + 2, buf1, sem1)
            return c
        lax.fori_loop(0, N_SPARSE // 2, body, 0)

        for j in range(NBLK):
            pltpu.sync_copy(
                emb_v.at[j],
                out_hbm.at[pl.ds(wid * BPW, BPW), pl.ds(j * 128, 128)],
            )

    return gather_kernel(tab128, row3, sub3)


def _tc_repack(tables):
    S = 1568
    NJ = -(-RPF // S)

    def body(x_ref, o_ref):
        x3 = x_ref[0].reshape(S, PACK, EMBED)
        for k in range(PACK):
            o_ref[0, :, pl.ds(k * EMBED, EMBED)] = x3[:, k, :]

    out = pl.pallas_call(
        body,
        grid=(N_SPARSE, NJ),
        in_specs=[pl.BlockSpec((1, S * PACK, EMBED), lambda f, j: (f, j, 0))],
        out_specs=pl.BlockSpec((1, S, 128), lambda f, j: (f, j, 0)),
        out_shape=jax.ShapeDtypeStruct((N_SPARSE, RPF, 128), jnp.float32),
        compiler_params=pltpu.CompilerParams(
            dimension_semantics=("arbitrary", "arbitrary")),
    )(tables)
    return out.reshape(N_SPARSE * RPF, 128)


BLK = 1024


def _dense_body(xd_ref, xe_ref, w1d_ref, w1e_ref, b1_ref, g1_ref, bt1_ref,
                w2_ref, b2_ref, g2_ref, bt2_ref,
                wlind_ref, wline_ref, blin_ref, wout_ref, o_ref):
    inv = 1.0 / (1.0 + BN_EPS) ** 0.5
    xd = xd_ref[...]
    xe = xe_ref[...]
    lin = (jnp.dot(xd, wlind_ref[...], preferred_element_type=jnp.float32)
           + jnp.dot(xe, wline_ref[...], preferred_element_type=jnp.float32)
           + blin_ref[...])
    h = (jnp.dot(xd, w1d_ref[...], preferred_element_type=jnp.float32)
         + jnp.dot(xe, w1e_ref[...], preferred_element_type=jnp.float32)
         + b1_ref[...])
    h = jnp.maximum(h * (g1_ref[...] * inv) + bt1_ref[...], 0.0)
    h = jnp.dot(h, w2_ref[...], preferred_element_type=jnp.float32) + b2_ref[...]
    h = jnp.maximum(h * (g2_ref[...] * inv) + bt2_ref[...], 0.0)
    dnn = jnp.dot(h, wout_ref[...], preferred_element_type=jnp.float32)
    o_ref[...] = jax.nn.sigmoid(lin + dnn)


def _tc_dense(dense_input, emb, W1, b1, g1, bt1, W2, b2, g2, bt2,
              W_lin, b_lin, W_out):
    pad = jnp.zeros((D_PAD - D_EMB, H1), jnp.float32)
    padl = jnp.zeros((D_PAD - D_EMB, 1), jnp.float32)
    w1d, w1e = W1[:N_DENSE], jnp.concatenate([W1[N_DENSE:], pad], axis=0)
    wlind = W_lin[:N_DENSE]
    wline = jnp.concatenate([W_lin[N_DENSE:], padl], axis=0)
    row = lambda v: v.reshape(1, -1)
    grid = (BATCH // BLK,)
    full = lambda a: pl.BlockSpec(a.shape, lambda i: (0, 0))
    return pl.pallas_call(
        _dense_body,
        grid=grid,
        in_specs=[
            pl.BlockSpec((BLK, N_DENSE), lambda i: (i, 0)),
            pl.BlockSpec((BLK, D_PAD), lambda i: (i, 0)),
            full(w1d), full(w1e), full(row(b1)), full(row(g1)), full(row(bt1)),
            full(W2), full(row(b2)), full(row(g2)), full(row(bt2)),
            full(wlind), full(wline), full(row(b_lin)), full(W_out),
        ],
        out_specs=pl.BlockSpec((BLK, 1), lambda i: (i, 0)),
        out_shape=jax.ShapeDtypeStruct((BATCH, 1), jnp.float32),
        compiler_params=pltpu.CompilerParams(
            dimension_semantics=("arbitrary",)),
    )(dense_input, emb, w1d, w1e, row(b1), row(g1), row(bt1),
      W2, row(b2), row(g2), row(bt2), wlind, wline, row(b_lin), W_out)


def kernel(dense_input, sparse_input, tables, W_lin, b_lin,
           W1, b1, g1, bt1, W2, b2, g2, bt2, W_out):
    tab128 = _tc_repack(tables)
    sp_t = sparse_input.reshape(NW, BPW, N_SPARSE).transpose(0, 2, 1)
    frow = (jnp.arange(N_SPARSE, dtype=jnp.int32) * RPF)[None, :, None]
    row3 = sp_t // PACK + frow
    sub3 = sp_t % PACK
    emb = _sc_gather(tab128, row3, sub3)
    return _tc_dense(dense_input, emb, W1, b1, g1, bt1, W2, b2, g2, bt2,
                     W_lin, b_lin, W_out)


# speedup vs baseline: 4.0051x; 4.0051x over previous
import functools

import jax
import jax.numpy as jnp
from jax import lax
from jax.experimental import pallas as pl
from jax.experimental.pallas import tpu as pltpu
from jax.experimental.pallas import tpu_sc as plsc

N_DENSE = 13
N_SPARSE = 26
VOCAB = 100000
EMBED = 16
BATCH = 4096
H1 = 256
H2 = 256
BN_EPS = 1e-3

NC = 2
NS = 16
NW = NC * NS
D_EMB = N_SPARSE * EMBED
CPW = D_EMB // NW


def _sc_embed_t(tab1d, idx_t):
    mesh = plsc.VectorSubcoreMesh(core_axis_name="c", subcore_axis_name="s")

    @functools.partial(
        pl.kernel,
        out_type=jax.ShapeDtypeStruct((D_EMB, BATCH), jnp.float32),
        mesh=mesh,
        scratch_types=[
            pltpu.VMEM((2, BATCH), jnp.int32),
            pltpu.VMEM((VOCAB,), jnp.float32),
            pltpu.VMEM((BATCH,), jnp.float32),
        ],
        compiler_params=pltpu.CompilerParams(
            needs_layout_passes=False, use_tc_tiling_on_sc=False),
    )
    def embed_kernel(tab_hbm, idx_hbm, out_hbm, idx_v, slab, col_v):
        wid = lax.axis_index("s") * NC + lax.axis_index("c")
        fe0 = wid * CPW
        f_lo = fe0 // EMBED
        f_hi = (fe0 + CPW - 1) // EMBED
        pltpu.sync_copy(idx_hbm.at[f_lo], idx_v.at[0])
        pltpu.sync_copy(idx_hbm.at[f_hi], idx_v.at[1])

        def col(jj, c):
            fe = fe0 + jj
            floc = fe // EMBED - f_lo
            pltpu.sync_copy(tab_hbm.at[pl.ds(fe * VOCAB, VOCAB)], slab)

            def grp(g, c2):
                iv = idx_v[floc, pl.ds(g * 16, 16)]
                col_v[pl.ds(g * 16, 16)] = plsc.load_gather(slab, [iv])
                return c2
            lax.fori_loop(0, BATCH // 16, grp, 0)
            pltpu.sync_copy(col_v, out_hbm.at[fe])
            return c
        lax.fori_loop(0, CPW, col, 0)

    return embed_kernel(tab1d, idx_t)


BLK = 1024


def _dense_body(xd_ref, xet_ref, w1d_ref, w1e_ref, b1_ref, g1_ref, bt1_ref,
                w2_ref, b2_ref, g2_ref, bt2_ref,
                wlind_ref, wline_ref, blin_ref, wout_ref, o_ref):
    inv = 1.0 / (1.0 + BN_EPS) ** 0.5
    cdim = (((0,), (0,)), ((), ()))
    xd = xd_ref[...]
    xet = xet_ref[...]
    lin = (jnp.dot(xd, wlind_ref[...], preferred_element_type=jnp.float32)
           + lax.dot_general(xet, wline_ref[...], cdim,
                             preferred_element_type=jnp.float32)
           + blin_ref[...])
    h = (jnp.dot(xd, w1d_ref[...], preferred_element_type=jnp.float32)
         + lax.dot_general(xet, w1e_ref[...], cdim,
                           preferred_element_type=jnp.float32)
         + b1_ref[...])
    h = jnp.maximum(h * (g1_ref[...] * inv) + bt1_ref[...], 0.0)
    h = jnp.dot(h, w2_ref[...], preferred_element_type=jnp.float32) + b2_ref[...]
    h = jnp.maximum(h * (g2_ref[...] * inv) + bt2_ref[...], 0.0)
    dnn = jnp.dot(h, wout_ref[...], preferred_element_type=jnp.float32)
    o_ref[...] = jax.nn.sigmoid(lin + dnn)


def _tc_dense(dense_input, emb_t, W1, b1, g1, bt1, W2, b2, g2, bt2,
              W_lin, b_lin, W_out):
    w1d, w1e = W1[:N_DENSE], W1[N_DENSE:]
    wlind, wline = W_lin[:N_DENSE], W_lin[N_DENSE:]
    row = lambda v: v.reshape(1, -1)
    grid = (BATCH // BLK,)
    full = lambda a: pl.BlockSpec(a.shape, lambda i: (0, 0))
    return pl.pallas_call(
        _dense_body,
        grid=grid,
        in_specs=[
            pl.BlockSpec((BLK, N_DENSE), lambda i: (i, 0)),
            pl.BlockSpec((D_EMB, BLK), lambda i: (0, i)),
            full(w1d), full(w1e), full(row(b1)), full(row(g1)), full(row(bt1)),
            full(W2), full(row(b2)), full(row(g2)), full(row(bt2)),
            full(wlind), full(wline), full(row(b_lin)), full(W_out),
        ],
        out_specs=pl.BlockSpec((BLK, 1), lambda i: (i, 0)),
        out_shape=jax.ShapeDtypeStruct((BATCH, 1), jnp.float32),
        compiler_params=pltpu.CompilerParams(
            dimension_semantics=("arbitrary",)),
    )(dense_input, emb_t, w1d, w1e, row(b1), row(g1), row(bt1),
      W2, row(b2), row(g2), row(bt2), wlind, wline, row(b_lin), W_out)


def kernel(dense_input, sparse_input, tables, W_lin, b_lin,
           W1, b1, g1, bt1, W2, b2, g2, bt2, W_out):
    tab1d = jnp.transpose(tables, (0, 2, 1)).reshape(N_SPARSE * EMBED * VOCAB)
    idx_t = sparse_input.T
    emb_t = _sc_embed_t(tab1d, idx_t)
    return _tc_dense(dense_input, emb_t, W1, b1, g1, bt1, W2, b2, g2, bt2,
                     W_lin, b_lin, W_out)


# speedup vs baseline: 6.0965x; 1.5222x over previous
import functools

import jax
import jax.numpy as jnp
from jax import lax
from jax.experimental import pallas as pl
from jax.experimental.pallas import tpu as pltpu
from jax.experimental.pallas import tpu_sc as plsc

N_DENSE = 13
N_SPARSE = 26
VOCAB = 100000
EMBED = 16
BATCH = 4096
H1 = 256
H2 = 256
BN_EPS = 1e-3

NC = 2
NS = 16
NW = NC * NS
D_EMB = N_SPARSE * EMBED
CPW = D_EMB // NW


def _sc_embed_t(tab1d, idx_t):
    mesh = plsc.VectorSubcoreMesh(core_axis_name="c", subcore_axis_name="s")

    @functools.partial(
        pl.kernel,
        out_type=jax.ShapeDtypeStruct((D_EMB, BATCH), jnp.float32),
        mesh=mesh,
        scratch_types=[
            pltpu.VMEM((2, BATCH), jnp.int32),
            pltpu.VMEM((VOCAB,), jnp.float32),
            pltpu.VMEM((BATCH,), jnp.float32),
        ],
        compiler_params=pltpu.CompilerParams(
            needs_layout_passes=False, use_tc_tiling_on_sc=False),
    )
    def embed_kernel(tab_hbm, idx_hbm, out_hbm, idx_v, slab, col_v):
        wid = lax.axis_index("s") * NC + lax.axis_index("c")
        fe0 = wid * CPW
        f_lo = fe0 // EMBED
        f_hi = (fe0 + CPW - 1) // EMBED
        pltpu.sync_copy(idx_hbm.at[f_lo], idx_v.at[0])
        pltpu.sync_copy(idx_hbm.at[f_hi], idx_v.at[1])

        def col(jj, c):
            fe = fe0 + jj
            floc = fe // EMBED - f_lo
            pltpu.sync_copy(tab_hbm.at[pl.ds(fe * VOCAB, VOCAB)], slab)

            def grp(g, c2):
                iv = idx_v[floc, pl.ds(g * 16, 16)]
                col_v[pl.ds(g * 16, 16)] = plsc.load_gather(slab, [iv])
                return c2
            lax.fori_loop(0, BATCH // 16, grp, 0)
            pltpu.sync_copy(col_v, out_hbm.at[fe])
            return c
        lax.fori_loop(0, CPW, col, 0)

    return embed_kernel(tab1d, idx_t)


def _tc_flatten(tab_t):
    def body(x_ref, o_ref):
        for f2 in range(2):
            for e in range(EMBED):
                o_ref[pl.ds((f2 * EMBED + e) * VOCAB, VOCAB)] = x_ref[f2, e, :]

    return pl.pallas_call(
        body,
        grid=(N_SPARSE // 2,),
        in_specs=[pl.BlockSpec((2, EMBED, VOCAB), lambda p: (p, 0, 0))],
        out_specs=pl.BlockSpec((2 * EMBED * VOCAB,), lambda p: (p,)),
        out_shape=jax.ShapeDtypeStruct((N_SPARSE * EMBED * VOCAB,),
                                       jnp.float32),
        compiler_params=pltpu.CompilerParams(
            dimension_semantics=("arbitrary",)),
    )(tab_t)


BLK = 1024


def _dense_body(xd_ref, xet_ref, w1d_ref, w1e_ref, b1_ref, g1_ref, bt1_ref,
                w2_ref, b2_ref, g2_ref, bt2_ref,
                wlind_ref, wline_ref, blin_ref, wout_ref, o_ref):
    inv = 1.0 / (1.0 + BN_EPS) ** 0.5
    cdim = (((0,), (0,)), ((), ()))
    xd = xd_ref[...]
    xet = xet_ref[...]
    lin = (jnp.dot(xd, wlind_ref[...], preferred_element_type=jnp.float32)
           + lax.dot_general(xet, wline_ref[...], cdim,
                             preferred_element_type=jnp.float32)
           + blin_ref[...])
    h = (jnp.dot(xd, w1d_ref[...], preferred_element_type=jnp.float32)
         + lax.dot_general(xet, w1e_ref[...], cdim,
                           preferred_element_type=jnp.float32)
         + b1_ref[...])
    h = jnp.maximum(h * (g1_ref[...] * inv) + bt1_ref[...], 0.0)
    h = jnp.dot(h, w2_ref[...], preferred_element_type=jnp.float32) + b2_ref[...]
    h = jnp.maximum(h * (g2_ref[...] * inv) + bt2_ref[...], 0.0)
    dnn = jnp.dot(h, wout_ref[...], preferred_element_type=jnp.float32)
    o_ref[...] = jax.nn.sigmoid(lin + dnn)


def _tc_dense(dense_input, emb_t, W1, b1, g1, bt1, W2, b2, g2, bt2,
              W_lin, b_lin, W_out):
    w1d, w1e = W1[:N_DENSE], W1[N_DENSE:]
    wlind, wline = W_lin[:N_DENSE], W_lin[N_DENSE:]
    row = lambda v: v.reshape(1, -1)
    grid = (BATCH // BLK,)
    full = lambda a: pl.BlockSpec(a.shape, lambda i: (0, 0))
    return pl.pallas_call(
        _dense_body,
        grid=grid,
        in_specs=[
            pl.BlockSpec((BLK, N_DENSE), lambda i: (i, 0)),
            pl.BlockSpec((D_EMB, BLK), lambda i: (0, i)),
            full(w1d), full(w1e), full(row(b1)), full(row(g1)), full(row(bt1)),
            full(W2), full(row(b2)), full(row(g2)), full(row(bt2)),
            full(wlind), full(wline), full(row(b_lin)), full(W_out),
        ],
        out_specs=pl.BlockSpec((BLK, 1), lambda i: (i, 0)),
        out_shape=jax.ShapeDtypeStruct((BATCH, 1), jnp.float32),
        compiler_params=pltpu.CompilerParams(
            dimension_semantics=("arbitrary",)),
    )(dense_input, emb_t, w1d, w1e, row(b1), row(g1), row(bt1),
      W2, row(b2), row(g2), row(bt2), wlind, wline, row(b_lin), W_out)


def kernel(dense_input, sparse_input, tables, W_lin, b_lin,
           W1, b1, g1, bt1, W2, b2, g2, bt2, W_out):
    tab1d = _tc_flatten(jnp.transpose(tables, (0, 2, 1)))
    idx_t = sparse_input.T
    emb_t = _sc_embed_t(tab1d, idx_t)
    return _tc_dense(dense_input, emb_t, W1, b1, g1, bt1, W2, b2, g2, bt2,
                     W_lin, b_lin, W_out)
